# baseline (speedup 1.0000x reference)
import jax
import jax.numpy as jnp
from jax import lax
from jax.experimental import pallas as pl
from jax.experimental.pallas import tpu as pltpu
from jax.experimental.pallas import tpu_sc as plsc

SIZE = 100000
RANK = 64
NNZ = 640000
BATCH = 16384

NTILES = 16
REP_PER_TILE = 6256
REP_PAD = NTILES * REP_PER_TILE
MARKER = BATCH
ACC_ROWS = BATCH + 16
ACC_WORDS = ACC_ROWS * RANK
ACC_WORDS_PER_TILE = ACC_WORDS // NTILES
B_PER_TILE = BATCH // NTILES
NNZ_ROWS = NNZ // 128
CHUNK_ROWS = 40
NCHUNKS = NNZ_ROWS // CHUNK_ROWS
CHUNK_ITERS = -(-NCHUNKS // NTILES)
FILL = 2048
GRP = 32
OUTW = 128


def _body(inp_h, rr_h, rc_h, rv_h, ir_h, ic_h, iv_h, out_re, out_im,
          rep, af, mb, zb, idx2, bv2, sb2, rows2, rows2b, cols2,
          cols2b, vals2, vals2b, f2, f2b, orow_a, orow_b,
          semA, semB, semC, semD):
    wid = lax.axis_index("s")
    cid = lax.axis_index("c")
    iota = lax.iota(jnp.int32, 16)

    def fill(i, c):
        mb[pl.ds(i * 16, 16)] = jnp.full((16,), MARKER, jnp.int32)
        zb[pl.ds(i * 16, 16)] = jnp.zeros((16,), jnp.float32)
        return c
    lax.fori_loop(0, FILL // 16, fill, None)

    rep_descs = []
    rep_base = pl.multiple_of(wid * REP_PER_TILE, 8)
    for k in range(3):
        rep_descs.append(pltpu.make_async_copy(
            mb, rep.at[pl.ds(rep_base + k * FILL, FILL)], semA))
    rtail = REP_PER_TILE - 3 * FILL
    rep_descs.append(pltpu.make_async_copy(
        mb.at[pl.ds(0, rtail)],
        rep.at[pl.ds(rep_base + 3 * FILL, rtail)], semA))
    for d in rep_descs:
        d.start()
    for d in rep_descs:
        d.wait()

    plsc.subcore_barrier()

    zero_descs = []
    ab = pl.multiple_of(wid * ACC_WORDS_PER_TILE, 8)
    for k in range(ACC_WORDS_PER_TILE // FILL):
        zero_descs.append(pltpu.make_async_copy(
            zb, af.at[pl.ds(pl.multiple_of(ab + k * FILL, 8), FILL)], semB))
    atail = ACC_WORDS_PER_TILE % FILL
    if atail:
        zero_descs.append(pltpu.make_async_copy(
            zb.at[pl.ds(0, atail)],
            af.at[pl.ds(pl.multiple_of(ab + ACC_WORDS_PER_TILE - atail, 8),
                        atail)], semB))
    for d in zero_descs:
        d.start()

    b0 = pl.multiple_of(wid * B_PER_TILE, 8)
    in_descs = [pltpu.make_async_copy(
        inp_h.at[pl.ds(b0 + j * 128, 128)], idx2.at[j], semA)
        for j in range(8)]
    for d in in_descs:
        d.start()
    for j in range(8):
        for i in range(8):
            bv2[j, pl.ds(i * 16, 16)] = (b0 + j * 128 + i * 16) + iota
    for d in in_descs:
        d.wait()
    sc_descs = [pltpu.make_async_copy(bv2.at[j], rep.at[idx2.at[j]], semA)
                for j in range(8)]
    for d in sc_descs:
        d.start()
    for d in sc_descs:
        d.wait()
    for d in zero_descs:
        d.wait()

    plsc.subcore_barrier()

    rowsS = (rows2, rows2b)
    colsS = (cols2, cols2b)
    valsS = (vals2, vals2b)
    fS = (f2, f2b)
    semL = (semA, semB)
    semS = (semC, semD)

    def accumulate(rows_h, cols_h, vals_h):
        def fire_loads(j):
            p = j & 1
            r0 = pl.multiple_of((wid + NTILES * j) * CHUNK_ROWS, 8)
            for d in [
                pltpu.make_async_copy(rows_h.at[pl.ds(r0, CHUNK_ROWS), :],
                                      rowsS[p], semL[p]),
                pltpu.make_async_copy(cols_h.at[pl.ds(r0, CHUNK_ROWS), :],
                                      colsS[p], semL[p]),
                pltpu.make_async_copy(vals_h.at[pl.ds(r0, CHUNK_ROWS), :],
                                      valsS[p], semL[p]),
            ]:
                d.start()

        def drain_loads(j):
            p = j & 1
            pltpu.make_async_copy(rows_h.at[pl.ds(0, CHUNK_ROWS), :],
                                  rowsS[p], semL[p]).wait()
            pltpu.make_async_copy(cols_h.at[pl.ds(0, CHUNK_ROWS), :],
                                  colsS[p], semL[p]).wait()
            pltpu.make_async_copy(vals_h.at[pl.ds(0, CHUNK_ROWS), :],
                                  valsS[p], semL[p]).wait()

        def fire_gathers(j):
            p = j & 1

            def gfire(k, c2):
                pltpu.async_copy(rep.at[rowsS[p].at[k]], fS[p].at[k],
                                 semL[p])
                return c2
            lax.fori_loop(0, CHUNK_ROWS, gfire, None)

        def drain_gathers(j):
            p = j & 1
            pltpu.make_async_copy(rows_h.at[pl.ds(0, CHUNK_ROWS), :],
                                  fS[p], semL[p]).wait()

        def compute(j):
            p = j & 1

            def comp(k, c2):
                for i in range(8):
                    sl = pl.ds(i * 16, 16)
                    cc = colsS[p][k, sl]
                    fS[p][k, sl] = fS[p][k, sl] * RANK + (cc & (RANK - 1))
                return c2
            lax.fori_loop(0, CHUNK_ROWS, comp, None)

        def fire_scatters(j):
            p = j & 1

            def sfire(k, c2):
                pltpu.async_copy(valsS[p].at[k], af.at[fS[p].at[k]],
                                 semS[p], add=True)
                return c2
            lax.fori_loop(0, CHUNK_ROWS, sfire, None)

        def drain_scatters(j):
            p = j & 1
            pltpu.make_async_copy(vals_h.at[pl.ds(0, CHUNK_ROWS), :],
                                  valsS[p], semS[p]).wait()

        def guarded(j, fn):
            if j == CHUNK_ITERS - 1:
                @pl.when((wid + NTILES * j) < NCHUNKS)
                def _():
                    fn(j)
            else:
                fn(j)

        guarded(0, fire_loads)
        guarded(0, drain_loads)
        guarded(0, fire_gathers)
        for j in range(CHUNK_ITERS):
            if j >= 1:
                guarded(j - 1, drain_scatters)
            if j + 1 < CHUNK_ITERS:
                guarded(j + 1, fire_loads)
            guarded(j, drain_gathers)
            guarded(j, compute)
            guarded(j, fire_scatters)
            if j + 1 < CHUNK_ITERS:
                guarded(j + 1, drain_loads)
                guarded(j + 1, fire_gathers)
        guarded(CHUNK_ITERS - 1, drain_scatters)

    @pl.when(cid == 0)
    def _re():
        accumulate(rr_h, rc_h, rv_h)

    @pl.when(cid == 1)
    def _im():
        accumulate(ir_h, ic_h, iv_h)

    plsc.subcore_barrier()

    for j in range(8):
        pltpu.sync_copy(rep.at[idx2.at[j]], sb2.at[j])

    rb = pl.multiple_of(wid * B_PER_TILE, 8)

    def emit(out_h):
        bufs = (orow_a, orow_b)
        fsem = (semA, semB)
        ssem = (semC, semD)
        NGRP = B_PER_TILE // GRP

        def fire(g, buf, fs):
            per_row = 128 // GRP
            jrow, half = g // per_row, (g % per_row) * GRP

            def grp(i, c):
                svec = sb2[jrow, pl.ds(half + i * 16, 16)]
                for l in range(16):
                    s = svec[l]
                    pltpu.async_copy(
                        af.at[pl.ds(pl.multiple_of(s * RANK, 8), RANK)],
                        buf.at[i * 16 + l, pl.ds(0, RANK)], fs)
                return c
            lax.fori_loop(0, GRP // 16, grp, None)

        store_descs = [None] * NGRP
        fire(0, bufs[0], fsem[0])
        for g in range(NGRP):
            p = g & 1
            pltpu.make_async_copy(rr_h.at[pl.ds(0, GRP // 2), :],
                                  f2.at[pl.ds(0, GRP // 2), :],
                                  fsem[p]).wait()
            if g + 1 < NGRP:
                if g >= 1:
                    store_descs[g - 1].wait()
                fire(g + 1, bufs[1 - p], fsem[1 - p])
            d = pltpu.make_async_copy(
                bufs[p], out_h.at[pl.ds(rb + g * GRP, GRP), :], ssem[p])
            d.start()
            store_descs[g] = d
        store_descs[NGRP - 2].wait()
        store_descs[NGRP - 1].wait()

    @pl.when(cid == 0)
    def _wre():
        emit(out_re)

    @pl.when(cid == 1)
    def _wim():
        emit(out_im)


@jax.jit
def _sc_call(inp, rr, rc, rv, ir, ic, iv):
    f32 = jnp.float32
    i32 = jnp.int32
    out_re, out_im = pl.kernel(
        _body,
        out_type=(jax.ShapeDtypeStruct((BATCH, OUTW), f32),
                  jax.ShapeDtypeStruct((BATCH, OUTW), f32)),
        mesh=plsc.VectorSubcoreMesh(core_axis_name="c", subcore_axis_name="s",
                                    num_cores=2, num_subcores=NTILES),
        scratch_types=[
            pltpu.VMEM_SHARED((REP_PAD,), i32),
            pltpu.VMEM_SHARED((ACC_WORDS,), f32),
            pltpu.VMEM((FILL,), i32),
            pltpu.VMEM((FILL,), f32),
            pltpu.VMEM((8, 128), i32),
            pltpu.VMEM((8, 128), i32),
            pltpu.VMEM((8, 128), i32),
            pltpu.VMEM((CHUNK_ROWS, 128), i32),
            pltpu.VMEM((CHUNK_ROWS, 128), i32),
            pltpu.VMEM((CHUNK_ROWS, 128), i32),
            pltpu.VMEM((CHUNK_ROWS, 128), i32),
            pltpu.VMEM((CHUNK_ROWS, 128), f32),
            pltpu.VMEM((CHUNK_ROWS, 128), f32),
            pltpu.VMEM((CHUNK_ROWS, 128), i32),
            pltpu.VMEM((CHUNK_ROWS, 128), i32),
            pltpu.VMEM((GRP, OUTW), f32),
            pltpu.VMEM((GRP, OUTW), f32),
            pltpu.SemaphoreType.DMA,
            pltpu.SemaphoreType.DMA,
            pltpu.SemaphoreType.DMA,
            pltpu.SemaphoreType.DMA,
        ],
    )(inp, rr, rc, rv, ir, ic, iv)
    return out_re[:, :RANK], out_im[:, :RANK]


def kernel(input, re_index, re_value, im_index, im_value):
    inp = input.astype(jnp.int32)
    rr = re_index[0].reshape(NNZ_ROWS, 128)
    rc = re_index[1].reshape(NNZ_ROWS, 128)
    rv = re_value.reshape(NNZ_ROWS, 128)
    ir = im_index[0].reshape(NNZ_ROWS, 128)
    ic = im_index[1].reshape(NNZ_ROWS, 128)
    iv = im_value.reshape(NNZ_ROWS, 128)
    return _sc_call(inp, rr, rc, rv, ir, ic, iv)


# speedup vs baseline: 1.0392x; 1.0392x over previous
import jax
import jax.numpy as jnp
from jax import lax
from jax.experimental import pallas as pl
from jax.experimental.pallas import tpu as pltpu
from jax.experimental.pallas import tpu_sc as plsc

SIZE = 100000
RANK = 64
NNZ = 640000
BATCH = 16384

NTILES = 16
REP_PER_TILE = 6256
REP_PAD = NTILES * REP_PER_TILE
MARKER = BATCH
ACC_ROWS = BATCH + 16
ACC_WORDS = ACC_ROWS * RANK
ACC_WORDS_PER_TILE = ACC_WORDS // NTILES
B_PER_TILE = BATCH // NTILES
NNZ_ROWS = NNZ // 128
CHUNK_ROWS = 40
NCHUNKS = NNZ_ROWS // CHUNK_ROWS
CHUNK_ITERS = -(-NCHUNKS // NTILES)
FILL = 2048
GRP = 64
OUTW = 128


def _body(inp_h, rr_h, rc_h, rv_h, ir_h, ic_h, iv_h, out_re, out_im,
          rep, af, mb, zb, idx2, bv2, sb2, rows2, cols2, vals2, s2, f2,
          orow_a, orow_b, semA, semB, semC, semD):
    wid = lax.axis_index("s")
    cid = lax.axis_index("c")
    iota = lax.iota(jnp.int32, 16)

    def fill(i, c):
        mb[pl.ds(i * 16, 16)] = jnp.full((16,), MARKER, jnp.int32)
        zb[pl.ds(i * 16, 16)] = jnp.zeros((16,), jnp.float32)
        return c
    lax.fori_loop(0, FILL // 16, fill, None)

    rep_descs = []
    rep_base = pl.multiple_of(wid * REP_PER_TILE, 8)
    for k in range(3):
        rep_descs.append(pltpu.make_async_copy(
            mb, rep.at[pl.ds(rep_base + k * FILL, FILL)], semA))
    rtail = REP_PER_TILE - 3 * FILL
    rep_descs.append(pltpu.make_async_copy(
        mb.at[pl.ds(0, rtail)],
        rep.at[pl.ds(rep_base + 3 * FILL, rtail)], semA))
    for d in rep_descs:
        d.start()
    for d in rep_descs:
        d.wait()

    plsc.subcore_barrier()

    zero_descs = []
    ab = pl.multiple_of(wid * ACC_WORDS_PER_TILE, 8)
    for k in range(ACC_WORDS_PER_TILE // FILL):
        zero_descs.append(pltpu.make_async_copy(
            zb, af.at[pl.ds(pl.multiple_of(ab + k * FILL, 8), FILL)], semB))
    atail = ACC_WORDS_PER_TILE % FILL
    if atail:
        zero_descs.append(pltpu.make_async_copy(
            zb.at[pl.ds(0, atail)],
            af.at[pl.ds(pl.multiple_of(ab + ACC_WORDS_PER_TILE - atail, 8),
                        atail)], semB))
    for d in zero_descs:
        d.start()

    b0 = pl.multiple_of(wid * B_PER_TILE, 8)
    in_descs = [pltpu.make_async_copy(
        inp_h.at[pl.ds(b0 + j * 128, 128)], idx2.at[j], semA)
        for j in range(8)]
    for d in in_descs:
        d.start()
    for j in range(8):
        for i in range(8):
            bv2[j, pl.ds(i * 16, 16)] = (b0 + j * 128 + i * 16) + iota
    for d in in_descs:
        d.wait()
    sc_descs = [pltpu.make_async_copy(bv2.at[j], rep.at[idx2.at[j]], semA)
                for j in range(8)]
    for d in sc_descs:
        d.start()
    for d in sc_descs:
        d.wait()
    for d in zero_descs:
        d.wait()

    plsc.subcore_barrier()

    def accumulate(rows_h, cols_h, vals_h):
        def chunk(jc, c):
            t = wid + NTILES * jc

            @pl.when(t < NCHUNKS)
            def _go():
                r0 = pl.multiple_of(t * CHUNK_ROWS, 8)
                dr = pltpu.make_async_copy(
                    rows_h.at[pl.ds(r0, CHUNK_ROWS), :], rows2, semA)
                dc = pltpu.make_async_copy(
                    cols_h.at[pl.ds(r0, CHUNK_ROWS), :], cols2, semA)
                dv = pltpu.make_async_copy(
                    vals_h.at[pl.ds(pl.multiple_of(t * CHUNK_ROWS * 128, 128),
                                    CHUNK_ROWS * 128)], vals2, semA)
                dr.start(); dc.start(); dv.start()
                dr.wait(); dc.wait(); dv.wait()

                def gfire(k, c2):
                    pltpu.async_copy(rep.at[rows2.at[k]], s2.at[k], semA)
                    return c2
                lax.fori_loop(0, CHUNK_ROWS, gfire, None)
                pltpu.make_async_copy(
                    rows_h.at[pl.ds(0, CHUNK_ROWS), :], s2, semA).wait()

                def comp(k, c2):
                    for i in range(8):
                        s = s2[k, pl.ds(i * 16, 16)]
                        cc = cols2[k, pl.ds(i * 16, 16)]
                        f2[k, pl.ds(i * 16, 16)] = s * RANK + (cc & (RANK - 1))
                    return c2
                lax.fori_loop(0, CHUNK_ROWS, comp, None)

                def sfire(k, c2):
                    pltpu.async_copy(
                        vals2.at[pl.ds(pl.multiple_of(k * 128, 8), 128)],
                        af.at[f2.at[k]], semA, add=True)
                    return c2
                lax.fori_loop(0, CHUNK_ROWS, sfire, None)
                pltpu.make_async_copy(
                    vals_h.at[pl.ds(0, CHUNK_ROWS * 128)], vals2,
                    semA).wait()
            return c
        lax.fori_loop(0, CHUNK_ITERS, chunk, None)

    @pl.when(cid == 0)
    def _re():
        accumulate(rr_h, rc_h, rv_h)

    @pl.when(cid == 1)
    def _im():
        accumulate(ir_h, ic_h, iv_h)

    plsc.subcore_barrier()

    for j in range(8):
        pltpu.sync_copy(rep.at[idx2.at[j]], sb2.at[j])

    rb = pl.multiple_of(wid * B_PER_TILE, 8)

    def emit(out_h):
        bufs = (orow_a, orow_b)
        fsem = (semA, semB)
        ssem = (semC, semD)
        NGRP = B_PER_TILE // GRP

        def fire(g, buf, fs):
            jrow, half = g >> 1, (g & 1) * GRP

            def grp(i, c):
                svec = sb2[jrow, pl.ds(half + i * 16, 16)]
                for l in range(16):
                    s = svec[l]
                    pltpu.async_copy(
                        af.at[pl.ds(pl.multiple_of(s * RANK, 8), RANK)],
                        buf.at[i * 16 + l, pl.ds(0, RANK)], fs)
                return c
            lax.fori_loop(0, GRP // 16, grp, None)

        store_descs = [None] * NGRP
        fire(0, bufs[0], fsem[0])
        for g in range(NGRP):
            p = g & 1
            pltpu.make_async_copy(rr_h.at[pl.ds(0, 32), :],
                                  f2.at[pl.ds(0, 32), :], fsem[p]).wait()
            if g + 1 < NGRP:
                if g >= 1:
                    store_descs[g - 1].wait()
                fire(g + 1, bufs[1 - p], fsem[1 - p])
            d = pltpu.make_async_copy(
                bufs[p], out_h.at[pl.ds(rb + g * GRP, GRP), :], ssem[p])
            d.start()
            store_descs[g] = d
        store_descs[NGRP - 2].wait()
        store_descs[NGRP - 1].wait()

    @pl.when(cid == 0)
    def _wre():
        emit(out_re)

    @pl.when(cid == 1)
    def _wim():
        emit(out_im)


@jax.jit
def _sc_call(inp, rr, rc, rv, ir, ic, iv):
    f32 = jnp.float32
    i32 = jnp.int32
    out_re, out_im = pl.kernel(
        _body,
        out_type=(jax.ShapeDtypeStruct((BATCH, OUTW), f32),
                  jax.ShapeDtypeStruct((BATCH, OUTW), f32)),
        mesh=plsc.VectorSubcoreMesh(core_axis_name="c", subcore_axis_name="s",
                                    num_cores=2, num_subcores=NTILES),
        scratch_types=[
            pltpu.VMEM_SHARED((REP_PAD,), i32),
            pltpu.VMEM_SHARED((ACC_WORDS,), f32),
            pltpu.VMEM((FILL,), i32),
            pltpu.VMEM((FILL,), f32),
            pltpu.VMEM((8, 128), i32),
            pltpu.VMEM((8, 128), i32),
            pltpu.VMEM((8, 128), i32),
            pltpu.VMEM((CHUNK_ROWS, 128), i32),
            pltpu.VMEM((CHUNK_ROWS, 128), i32),
            pltpu.VMEM((CHUNK_ROWS * 128,), f32),
            pltpu.VMEM((CHUNK_ROWS, 128), i32),
            pltpu.VMEM((CHUNK_ROWS, 128), i32),
            pltpu.VMEM((GRP, OUTW), f32),
            pltpu.VMEM((GRP, OUTW), f32),
            pltpu.SemaphoreType.DMA,
            pltpu.SemaphoreType.DMA,
            pltpu.SemaphoreType.DMA,
            pltpu.SemaphoreType.DMA,
        ],
    )(inp, rr, rc, rv, ir, ic, iv)
    return out_re[:, :RANK], out_im[:, :RANK]


def kernel(input, re_index, re_value, im_index, im_value):
    inp = input.astype(jnp.int32)
    rr = re_index[0].reshape(NNZ_ROWS, 128)
    rc = re_index[1].reshape(NNZ_ROWS, 128)
    rv = re_value
    ir = im_index[0].reshape(NNZ_ROWS, 128)
    ic = im_index[1].reshape(NNZ_ROWS, 128)
    iv = im_value
    return _sc_call(inp, rr, rc, rv, ir, ic, iv)
